# baseline (speedup 1.0000x reference)
import functools

import jax
import jax.numpy as jnp
from jax.experimental import pallas as pl
from jax.experimental.pallas import tpu as pltpu


def _vq_tile_kernel(x_ref, ea_ref, et_ref, q_ref, partial_ref, *, imgs):
    ea = ea_ref[...]
    et = et_ref[...]
    s128 = None
    for im in range(imgs):
        x4 = x_ref[im]
        d, h, w = x4.shape
        t = h * w
        x = x4.reshape(d, t)

        ones2 = (jax.lax.broadcasted_iota(jnp.int32, (8, t), 0) < 2
                 ).astype(x.dtype)
        xa = jnp.concatenate([x, ones2], axis=0)
        dist = jax.lax.dot_general(
            ea.astype(jnp.bfloat16), xa.astype(jnp.bfloat16),
            dimension_numbers=(((1,), (0,)), ((), ())),
            preferred_element_type=jnp.float32)

        min_d = jnp.min(dist, axis=0, keepdims=True)
        one_hot = (dist == min_d).astype(jnp.bfloat16)

        qc = jax.lax.dot_general(
            et.astype(jnp.bfloat16), one_hot,
            dimension_numbers=(((1,), (0,)), ((), ())),
            preferred_element_type=jnp.float32)
        q = qc[:d]
        count = qc[d:d + 1]
        q = q * jnp.where(count > 1.0, 1.0 / count, 1.0)
        q_ref[im] = q.reshape(d, h, w).astype(q_ref.dtype)

        d2 = (q - x) * (q - x)
        s8 = d2[:8]
        for r in range(8, d, 8):
            s8 = s8 + d2[r:r + 8]
        for c in range(0, t, 128):
            blk = s8[:, c:c + 128]
            s128 = blk if s128 is None else s128 + blk
    partial_ref[...] = s128


def kernel(latents_nchw, embedding, beta=0.25):
    B, D, H, W = latents_nchw.shape
    K, D2 = embedding.shape
    assert D == D2
    HW = H * W
    N = B * HW

    e32 = embedding.astype(jnp.float32)
    half_e2 = 0.5 * jnp.sum(e32 * e32, axis=1)
    hh_hi = half_e2.astype(jnp.bfloat16).astype(jnp.float32)
    hh_lo = half_e2 - hh_hi
    ea = jnp.concatenate(
        [-e32, hh_hi[:, None], hh_lo[:, None],
         jnp.zeros((K, 6), jnp.float32)], axis=1)
    et = jnp.concatenate(
        [e32.T, jnp.ones((1, K), jnp.float32),
         jnp.zeros((7, K), jnp.float32)], axis=0)

    cost = pl.CostEstimate(
        flops=4 * N * K * (D + 8),
        transcendentals=0,
        bytes_accessed=2 * N * D * 4 + 2 * K * (D + 8) * 4
        + B * 8 * 128 * 4,
    )

    imgs = 4 if B % 4 == 0 else (2 if B % 2 == 0 else 1)
    nsteps = B // imgs
    q4, partials = pl.pallas_call(
        functools.partial(_vq_tile_kernel, imgs=imgs),
        out_shape=(
            jax.ShapeDtypeStruct((B, D, H, W), latents_nchw.dtype),
            jax.ShapeDtypeStruct((nsteps * 8, 128), jnp.float32),
        ),
        grid=(nsteps,),
        in_specs=[
            pl.BlockSpec((imgs, D, H, W), lambda b: (b, 0, 0, 0)),
            pl.BlockSpec((K, D + 8), lambda b: (0, 0)),
            pl.BlockSpec((D + 8, K), lambda b: (0, 0)),
        ],
        out_specs=[
            pl.BlockSpec((imgs, D, H, W), lambda b: (b, 0, 0, 0)),
            pl.BlockSpec((8, 128), lambda b: (b, 0)),
        ],
        compiler_params=pltpu.CompilerParams(
            dimension_semantics=("parallel",),
            vmem_limit_bytes=60 << 20,
        ),
        cost_estimate=cost,
    )(latents_nchw, ea, et)

    sse = jnp.sum(partials)
    mse = sse / jnp.float32(N * D)
    vq_loss = beta * mse + mse
    return q4, vq_loss


# speedup vs baseline: 1.0257x; 1.0257x over previous
import functools

import jax
import jax.numpy as jnp
from jax.experimental import pallas as pl
from jax.experimental.pallas import tpu as pltpu


def _vq_tile_kernel(x_ref, ea_ref, et_ref, q_ref, partial_ref, *, imgs):
    ea = ea_ref[...]
    et = et_ref[...]
    s128 = None
    for im in range(imgs):
        x4 = x_ref[im]
        d, h, w = x4.shape
        t = h * w
        x = x4.reshape(d, t)

        ones2 = (jax.lax.broadcasted_iota(jnp.int32, (8, t), 0) < 2
                 ).astype(x.dtype)
        xa = jnp.concatenate([x, ones2], axis=0)
        dist = jax.lax.dot_general(
            ea.astype(jnp.bfloat16), xa.astype(jnp.bfloat16),
            dimension_numbers=(((1,), (0,)), ((), ())),
            preferred_element_type=jnp.float32)

        min_d = jnp.min(dist, axis=0, keepdims=True)
        one_hot = (dist == min_d).astype(jnp.bfloat16)

        qc = jax.lax.dot_general(
            et.astype(jnp.bfloat16), one_hot,
            dimension_numbers=(((1,), (0,)), ((), ())),
            preferred_element_type=jnp.float32)
        q = qc[:d]
        count = qc[d:d + 1]
        q = q * jnp.where(count > 1.0, 1.0 / count, 1.0)
        q_ref[im] = q.reshape(d, h, w).astype(q_ref.dtype)

        d2 = (q - x) * (q - x)
        s8 = d2[:8]
        for r in range(8, d, 8):
            s8 = s8 + d2[r:r + 8]
        for c in range(0, t, 128):
            blk = s8[:, c:c + 128]
            s128 = blk if s128 is None else s128 + blk
    partial_ref[...] = s128


def kernel(latents_nchw, embedding, beta=0.25):
    B, D, H, W = latents_nchw.shape
    K, D2 = embedding.shape
    assert D == D2
    HW = H * W
    N = B * HW

    e32 = embedding.astype(jnp.float32)
    half_e2 = 0.5 * jnp.sum(e32 * e32, axis=1)
    hh_hi = half_e2.astype(jnp.bfloat16).astype(jnp.float32)
    hh_lo = half_e2 - hh_hi
    ea = jnp.concatenate(
        [-e32, hh_hi[:, None], hh_lo[:, None],
         jnp.zeros((K, 6), jnp.float32)], axis=1)
    et = jnp.concatenate(
        [e32.T, jnp.ones((1, K), jnp.float32),
         jnp.zeros((7, K), jnp.float32)], axis=0)

    cost = pl.CostEstimate(
        flops=4 * N * K * (D + 8),
        transcendentals=0,
        bytes_accessed=2 * N * D * 4 + 2 * K * (D + 8) * 4
        + B * 8 * 128 * 4,
    )

    imgs = 2 if B % 2 == 0 else 1
    nsteps = B // imgs
    q4, partials = pl.pallas_call(
        functools.partial(_vq_tile_kernel, imgs=imgs),
        out_shape=(
            jax.ShapeDtypeStruct((B, D, H, W), latents_nchw.dtype),
            jax.ShapeDtypeStruct((nsteps * 8, 128), jnp.float32),
        ),
        grid=(nsteps,),
        in_specs=[
            pl.BlockSpec((imgs, D, H, W), lambda b: (b, 0, 0, 0)),
            pl.BlockSpec((K, D + 8), lambda b: (0, 0)),
            pl.BlockSpec((D + 8, K), lambda b: (0, 0)),
        ],
        out_specs=[
            pl.BlockSpec((imgs, D, H, W), lambda b: (b, 0, 0, 0)),
            pl.BlockSpec((8, 128), lambda b: (b, 0)),
        ],
        compiler_params=pltpu.CompilerParams(
            dimension_semantics=("parallel",),
            vmem_limit_bytes=60 << 20,
        ),
        cost_estimate=cost,
    )(latents_nchw, ea, et)

    sse = jnp.sum(partials)
    mse = sse / jnp.float32(N * D)
    vq_loss = beta * mse + mse
    return q4, vq_loss
